# Initial kernel scaffold; baseline (speedup 1.0000x reference)
import jax
import jax.numpy as jnp
from jax.experimental import pallas as pl


def kernel(features, in_w, in_b, out_w, out_b):
    raise NotImplementedError("write your pallas kernel here")


# speedup vs baseline: 2.0876x; 2.0876x over previous
import numpy as np
import jax
import jax.numpy as jnp
from jax.experimental import pallas as pl
from jax.experimental.pallas import tpu as pltpu

_EPS = 1e-08
_LOG_EPS = float(np.log(1e-08))
_N_SLOTS = 16


def _sb_kernel(feat_ref, inw_ref, inb_ref, outw_ref, outb_ref, lseed_ref,
               out_ref, g_ref, masks_ref):
    f = feat_ref[0]
    P = f.shape[0]

    proj = jax.lax.dot_general(f.astype(jnp.bfloat16),
                               inw_ref[...].astype(jnp.bfloat16),
                               (((1,), (1,)), ((), ())),
                               preferred_element_type=jnp.float32)
    proj = proj + inb_ref[...]
    norm = jnp.sqrt(jnp.sum(proj * proj, axis=-1, keepdims=True))
    proj = proj / jnp.maximum(norm, 1e-12)

    g_ref[...] = jax.lax.dot_general(proj, proj, (((1,), (1,)), ((), ())),
                                     preferred_element_type=jnp.float32)

    lseeds = lseed_ref[0]
    ids = jax.lax.broadcasted_iota(jnp.int32, (1, P), 1)
    log_scope = jnp.zeros((1, P), jnp.float32)

    for k in range(_N_SLOTS):
        v = log_scope + lseeds
        mx = jnp.max(v)
        idx = jnp.min(jnp.where(v == mx, ids, P))
        grow = g_ref[pl.ds(idx, 1), :]
        dists = 2.0 - 2.0 * grow
        log_alpha = jnp.maximum(-dists, _LOG_EPS)
        masks_ref[k:k + 1, :] = jnp.exp(log_scope + log_alpha)
        log_scope = log_scope + jnp.log(
            jnp.maximum(1.0 - jnp.exp(log_alpha), _EPS))

    masks = masks_ref[...]
    slots = jax.lax.dot_general(masks.astype(jnp.bfloat16),
                                f.astype(jnp.bfloat16),
                                (((1,), (0,)), ((), ())),
                                preferred_element_type=jnp.float32)
    msum = jnp.sum(masks, axis=1, keepdims=True)
    slots = slots / jnp.maximum(msum, _EPS)
    outv = jax.lax.dot_general(slots.astype(jnp.bfloat16),
                               outw_ref[...].astype(jnp.bfloat16),
                               (((1,), (1,)), ((), ())),
                               preferred_element_type=jnp.float32)
    out_ref[0] = outv + outb_ref[...]


def kernel(features, in_w, in_b, out_w, out_b):
    bs, P, D = features.shape
    O = out_w.shape[0]
    seeds = jax.random.uniform(jax.random.key(42), (bs, P), dtype=jnp.float32)
    log_seeds = jnp.log(jnp.clip(seeds, _EPS, None)).reshape(bs, 1, P)
    in_b2 = in_b.reshape(1, D)
    out_b2 = out_b.reshape(1, O)
    return pl.pallas_call(
        _sb_kernel,
        grid=(bs,),
        in_specs=[
            pl.BlockSpec((1, P, D), lambda b: (b, 0, 0)),
            pl.BlockSpec((D, D), lambda b: (0, 0)),
            pl.BlockSpec((1, D), lambda b: (0, 0)),
            pl.BlockSpec((O, D), lambda b: (0, 0)),
            pl.BlockSpec((1, O), lambda b: (0, 0)),
            pl.BlockSpec((1, 1, P), lambda b: (b, 0, 0)),
        ],
        out_specs=pl.BlockSpec((1, _N_SLOTS, O), lambda b: (b, 0, 0)),
        out_shape=jax.ShapeDtypeStruct((bs, _N_SLOTS, O), jnp.float32),
        scratch_shapes=[
            pltpu.VMEM((P, P), jnp.float32),
            pltpu.VMEM((_N_SLOTS, P), jnp.float32),
        ],
        compiler_params=pltpu.CompilerParams(
            dimension_semantics=("parallel",)),
    )(features, in_w, in_b2, out_w, out_b2, log_seeds)


# speedup vs baseline: 2.9651x; 1.4203x over previous
import numpy as np
import jax
import jax.numpy as jnp
from jax.experimental import pallas as pl
from jax.experimental.pallas import tpu as pltpu

_EPS = 1e-08
_LOG_EPS = float(np.log(1e-08))
_N_SLOTS = 16
_S = 2


def _sb_kernel(feat_ref, inw_ref, inb_ref, outw_ref, outb_ref, lseed_ref,
               out_ref, g_ref, masks_ref):
    P = feat_ref.shape[1]
    inw_bf = inw_ref[...].astype(jnp.bfloat16)
    ids = jax.lax.broadcasted_iota(jnp.int32, (1, P), 1)

    fs, projs = [], []
    for s in range(_S):
        f = feat_ref[s]
        fs.append(f)
        proj = jax.lax.dot_general(f.astype(jnp.bfloat16), inw_bf,
                                   (((1,), (1,)), ((), ())),
                                   preferred_element_type=jnp.float32)
        proj = proj + inb_ref[...]
        norm = jnp.sqrt(jnp.sum(proj * proj, axis=-1, keepdims=True))
        proj = proj / jnp.maximum(norm, 1e-12)
        projs.append(proj)
        g_ref[s] = jax.lax.dot_general(proj, proj, (((1,), (1,)), ((), ())),
                                       preferred_element_type=jnp.float32)

    scopes = [jnp.zeros((1, P), jnp.float32)] * _S
    for k in range(_N_SLOTS):
        for s in range(_S):
            log_scope = scopes[s]
            v = log_scope + lseed_ref[s]
            mx = jnp.max(v)
            idx = jnp.min(jnp.where(v == mx, ids, P))
            grow = g_ref[s, pl.ds(idx, 1), :]
            dists = 2.0 - 2.0 * grow
            log_alpha = jnp.maximum(-dists, _LOG_EPS)
            masks_ref[s, k:k + 1, :] = jnp.exp(log_scope + log_alpha)
            scopes[s] = log_scope + jnp.log(
                jnp.maximum(1.0 - jnp.exp(log_alpha), _EPS))

    outw_bf = outw_ref[...].astype(jnp.bfloat16)
    for s in range(_S):
        masks = masks_ref[s]
        slots = jax.lax.dot_general(masks.astype(jnp.bfloat16),
                                    fs[s].astype(jnp.bfloat16),
                                    (((1,), (0,)), ((), ())),
                                    preferred_element_type=jnp.float32)
        msum = jnp.sum(masks, axis=1, keepdims=True)
        slots = slots / jnp.maximum(msum, _EPS)
        outv = jax.lax.dot_general(slots.astype(jnp.bfloat16), outw_bf,
                                   (((1,), (1,)), ((), ())),
                                   preferred_element_type=jnp.float32)
        out_ref[s] = outv + outb_ref[...]


def kernel(features, in_w, in_b, out_w, out_b):
    bs, P, D = features.shape
    O = out_w.shape[0]
    seeds = jax.random.uniform(jax.random.key(42), (bs, P), dtype=jnp.float32)
    log_seeds = jnp.log(jnp.clip(seeds, _EPS, None)).reshape(bs, 1, P)
    in_b2 = in_b.reshape(1, D)
    out_b2 = out_b.reshape(1, O)
    return pl.pallas_call(
        _sb_kernel,
        grid=(bs // _S,),
        in_specs=[
            pl.BlockSpec((_S, P, D), lambda b: (b, 0, 0)),
            pl.BlockSpec((D, D), lambda b: (0, 0)),
            pl.BlockSpec((1, D), lambda b: (0, 0)),
            pl.BlockSpec((O, D), lambda b: (0, 0)),
            pl.BlockSpec((1, O), lambda b: (0, 0)),
            pl.BlockSpec((_S, 1, P), lambda b: (b, 0, 0)),
        ],
        out_specs=pl.BlockSpec((_S, _N_SLOTS, O), lambda b: (b, 0, 0)),
        out_shape=jax.ShapeDtypeStruct((bs, _N_SLOTS, O), jnp.float32),
        scratch_shapes=[
            pltpu.VMEM((_S, P, P), jnp.float32),
            pltpu.VMEM((_S, _N_SLOTS, P), jnp.float32),
        ],
        compiler_params=pltpu.CompilerParams(
            dimension_semantics=("parallel",)),
    )(features, in_w, in_b2, out_w, out_b2, log_seeds)
